# baseline (speedup 1.0000x reference)
import functools

import jax
import jax.numpy as jnp
from jax import lax
from jax.experimental import pallas as pl
from jax.experimental.pallas import tpu as pltpu
from jax.experimental.pallas import tpu_sc as plsc

_N = 10000
_D = 128
_H = 5
_C = 128
_K = 16

_NPAD = 10240
_RB = 400
_WT = 2048
_NTILES = _NPAD // _WT
_NBLK = _N // _RB

_INF = 1e30
_BIGI = 1 << 30


_NLVL = 4


def _transform_body(xb_ref, wl_ref, bl_ref, wr_ref, br_ref,
                    xl_ref, xr_ref):
  xb = xb_ref[...]
  xl_ref[...] = (
      jnp.dot(xb, wl_ref[...], preferred_element_type=jnp.float32)
      + bl_ref[...])
  xr_ref[...] = (
      jnp.dot(xb, wr_ref[...], preferred_element_type=jnp.float32)
      + br_ref[...])


def _transforms(x, Wl, bl, Wr, br):
  return pl.pallas_call(
      _transform_body,
      grid=(_NBLK,),
      in_specs=[
          pl.BlockSpec((_RB, _D), lambda i: (i, 0)),
          pl.BlockSpec((_D, _H * _C), lambda i: (0, 0)),
          pl.BlockSpec((1, _H * _C), lambda i: (0, 0)),
          pl.BlockSpec((_D, _H * _C), lambda i: (0, 0)),
          pl.BlockSpec((1, _H * _C), lambda i: (0, 0)),
      ],
      out_specs=[
          pl.BlockSpec((_RB, _H * _C), lambda i: (i, 0)),
          pl.BlockSpec((_RB, _H * _C), lambda i: (i, 0)),
      ],
      out_shape=[
          jax.ShapeDtypeStruct((_N, _H * _C), jnp.float32),
          jax.ShapeDtypeStruct((_N, _H * _C), jnp.float32),
      ],
  )(x, Wl, bl.reshape(1, -1), Wr, br.reshape(1, -1))


def _make_knn_body(off_blocks):
 def _knn_body(xb_ref, xcols_ref, nbr_ref):
  rb = pl.program_id(0) + off_blocks
  xb = xb_ref[...]

  M = [jnp.full((_RB, 128), _INF, jnp.float32) for _ in range(_NLVL)]
  I = [jnp.zeros((_RB, 128), jnp.int32) for _ in range(_NLVL)]

  rows_col = lax.broadcasted_iota(jnp.int32, (_RB, 128), 0) + rb * _RB
  lane_col = lax.broadcasted_iota(jnp.int32, (_RB, 128), 1)

  for t in range(_NTILES):
    xc = xcols_ref[t]
    ones8 = jnp.ones((8, _D), jnp.float32)
    sq = lax.dot_general(ones8, xc * xc, (((1,), (1,)), ((), ())),
                         preferred_element_type=jnp.float32,
                         precision=lax.Precision.HIGHEST)[0:1, :]
    colrow = (lax.broadcasted_iota(jnp.int32, (1, _WT), 1) + t * _WT)
    sq = jnp.where(colrow >= _N, _INF, sq)

    dots = lax.dot_general(xb, xc, (((1,), (1,)), ((), ())),
                           preferred_element_type=jnp.float32)
    s = (sq - 2.0 * dots).reshape(_RB, _WT // 128, 128)

    for g in range(_WT // 128):
      v = s[:, g, :]
      iv = lane_col + (t * _WT + g * 128)
      v = jnp.where(iv == rows_col, _INF, v)
      for l in range(_NLVL):
        cond = v < M[l]
        if l < _NLVL - 1:
          M[l], v = jnp.where(cond, v, M[l]), jnp.where(cond, M[l], v)
          I[l], iv = jnp.where(cond, iv, I[l]), jnp.where(cond, I[l], iv)
        else:
          M[l] = jnp.where(cond, v, M[l])
          I[l] = jnp.where(cond, iv, I[l])

  cand = jnp.concatenate(M, axis=1)
  icand = jnp.concatenate(I, axis=1)
  for k in range(_K):
    m = jnp.min(cand, axis=1, keepdims=True)
    hit = cand <= m
    amin = jnp.min(jnp.where(hit, icand, _BIGI), axis=1, keepdims=True)
    nbr_ref[:, k:k + 1] = amin
    cand = jnp.where(hit, _INF, cand)
 return _knn_body


def _knn_topk(x, xcols, off_blocks, nblk):
  return pl.pallas_call(
      _make_knn_body(off_blocks),
      grid=(nblk,),
      in_specs=[
          pl.BlockSpec((_RB, _D), lambda i, o=off_blocks: (i + o, 0)),
          pl.BlockSpec((_NTILES, _WT, _D), lambda i: (0, 0, 0)),
      ],
      out_specs=pl.BlockSpec((_RB, _K), lambda i: (i, 0)),
      out_shape=jax.ShapeDtypeStruct((nblk * _RB, _K), jnp.int32),
  )(x, xcols)


_NW = 32
_CG = 40


def _gather_src(nbr_flat, xl):
  _E = nbr_flat.shape[0]
  _EPW = _E // _NW
  _NCHUNK = _EPW // _CG
  mesh = plsc.VectorSubcoreMesh(core_axis_name="c", subcore_axis_name="s")

  @functools.partial(
      pl.kernel,
      out_type=jax.ShapeDtypeStruct((_E, _H * _C), jnp.float32),
      mesh=mesh,
      scratch_types=[
          pltpu.VMEM((2, _CG), jnp.int32),
          pltpu.VMEM((2, _CG, _H * _C), jnp.float32),
          pltpu.SemaphoreType.DMA,
          pltpu.SemaphoreType.DMA,
      ],
  )
  def gather_kernel(idx_hbm, xl_hbm, out_hbm, idx_v, rows_v, sem0, sem1):
    wid = lax.axis_index("s") * 2 + lax.axis_index("c")
    base = wid * _EPW
    sems = [sem0, sem1]

    pltpu.sync_copy(idx_hbm.at[pl.ds(base, _CG)], idx_v.at[0])
    pltpu.async_copy(xl_hbm.at[idx_v.at[0]], rows_v.at[0], sems[0])

    def body(j, _):
      for b in range(2):
        i = 2 * j + b

        @pl.when(i + 1 < _NCHUNK)
        def _start_next():
          pltpu.sync_copy(idx_hbm.at[pl.ds(base + (i + 1) * _CG, _CG)],
                          idx_v.at[1 - b])
          pltpu.async_copy(xl_hbm.at[idx_v.at[1 - b]], rows_v.at[1 - b],
                           sems[1 - b])

        @pl.when(i < _NCHUNK)
        def _drain_cur():
          pltpu.make_async_copy(xl_hbm.at[idx_v.at[b]], rows_v.at[b],
                                sems[b]).wait()
          pltpu.sync_copy(rows_v.at[b],
                          out_hbm.at[pl.ds(base + i * _CG, _CG)])

      return 0

    lax.fori_loop(0, (_NCHUNK + 1) // 2, body, 0)

  return gather_kernel(nbr_flat, xl)


_RA = 400


def _attn_body(src_ref, xr_ref, attw_ref, bias_ref, out_ref):
  src = src_ref[...]
  xr = xr_ref[...]
  xr_rep = jnp.broadcast_to(
      xr[:, None, :], (_RA, _K, _H * _C)).reshape(_RA * _K, _H * _C)
  e = src + xr_rep
  e = jnp.maximum(e, 0.2 * e)
  lg = jnp.dot(e, attw_ref[...],
               preferred_element_type=jnp.float32)
  lg3 = lg.reshape(_RA, _K, _C)
  m = jnp.max(lg3, axis=1, keepdims=True)
  p = jnp.exp(lg3 - m)
  al = p / jnp.sum(p, axis=1, keepdims=True)
  src3 = src.reshape(_RA, _K, _H * _C)
  acc = jnp.zeros((_RA, _C), jnp.float32)
  for h in range(_H):
    alh = al[:, :, h:h + 1]
    acc = acc + jnp.sum(alh * src3[:, :, h * _C:(h + 1) * _C], axis=1)
  out_ref[...] = acc * (1.0 / _H) + bias_ref[...]


def _attention(src, xr, attw, bias):
  nrows = xr.shape[0]
  return pl.pallas_call(
      _attn_body,
      grid=(nrows // _RA,),
      in_specs=[
          pl.BlockSpec((_RA * _K, _H * _C), lambda i: (i, 0)),
          pl.BlockSpec((_RA, _H * _C), lambda i: (i, 0)),
          pl.BlockSpec((_H * _C, _C), lambda i: (0, 0)),
          pl.BlockSpec((1, _C), lambda i: (0, 0)),
      ],
      out_specs=pl.BlockSpec((_RA, _C), lambda i: (i, 0)),
      out_shape=jax.ShapeDtypeStruct((nrows, _C), jnp.float32),
  )(src, xr, attw, bias.reshape(1, -1))


_PARTS = (8, 8, 9)


def kernel(x, Wl, bl, Wr, br, att, bias):
  xl, xr = _transforms(x, Wl, bl, Wr, br)
  xcols = jnp.pad(x, ((0, _NPAD - _N), (0, 0))).reshape(_NTILES, _WT, _D)
  attw = jnp.zeros((_H * _C, _C), jnp.float32).at[
      jnp.arange(_H * _C), jnp.arange(_H * _C) // _C].set(att.reshape(-1))

  outs = []
  off = 0
  srcs = []
  for nblk in _PARTS:
    nbr = _knn_topk(x, xcols, off, nblk)
    srcs.append(_gather_src(nbr.reshape(-1), xl))
    off += nblk
  off = 0
  for nblk, src in zip(_PARTS, srcs):
    outs.append(_attention(
        src, xr[off * _RB:(off + nblk) * _RB], attw, bias))
    off += nblk
  return jnp.concatenate(outs, axis=0)


# speedup vs baseline: 1.0013x; 1.0013x over previous
import functools

import jax
import jax.numpy as jnp
from jax import lax
from jax.experimental import pallas as pl
from jax.experimental.pallas import tpu as pltpu
from jax.experimental.pallas import tpu_sc as plsc

_N = 10000
_D = 128
_H = 5
_C = 128
_K = 16

_NPAD = 10240
_RB = 400
_WT = 2048
_NTILES = _NPAD // _WT
_NBLK = _N // _RB

_INF = 1e30
_BIGI = 1 << 30


_NLVL = 4


def _transform_body(xb_ref, wl_ref, bl_ref, wr_ref, br_ref,
                    xl_ref, xr_ref):
  xb = xb_ref[...]
  xl_ref[...] = (
      jnp.dot(xb, wl_ref[...], preferred_element_type=jnp.float32)
      + bl_ref[...])
  xr_ref[...] = (
      jnp.dot(xb, wr_ref[...], preferred_element_type=jnp.float32)
      + br_ref[...])


def _transforms(x, Wl, bl, Wr, br):
  return pl.pallas_call(
      _transform_body,
      grid=(_NBLK,),
      in_specs=[
          pl.BlockSpec((_RB, _D), lambda i: (i, 0)),
          pl.BlockSpec((_D, _H * _C), lambda i: (0, 0)),
          pl.BlockSpec((1, _H * _C), lambda i: (0, 0)),
          pl.BlockSpec((_D, _H * _C), lambda i: (0, 0)),
          pl.BlockSpec((1, _H * _C), lambda i: (0, 0)),
      ],
      out_specs=[
          pl.BlockSpec((_RB, _H * _C), lambda i: (i, 0)),
          pl.BlockSpec((_RB, _H * _C), lambda i: (i, 0)),
      ],
      out_shape=[
          jax.ShapeDtypeStruct((_N, _H * _C), jnp.float32),
          jax.ShapeDtypeStruct((_N, _H * _C), jnp.float32),
      ],
  )(x, Wl, bl.reshape(1, -1), Wr, br.reshape(1, -1))


def _make_knn_body(off_blocks):
 def _knn_body(xb_ref, xcols_ref, nbr_ref):
  rb = pl.program_id(0) + off_blocks
  xb = xb_ref[...]

  M = [jnp.full((_RB, 128), _INF, jnp.float32) for _ in range(_NLVL)]
  I = [jnp.zeros((_RB, 128), jnp.int32) for _ in range(_NLVL)]

  rows_col = lax.broadcasted_iota(jnp.int32, (_RB, 128), 0) + rb * _RB
  lane_col = lax.broadcasted_iota(jnp.int32, (_RB, 128), 1)

  for t in range(_NTILES):
    xc = xcols_ref[t]
    ones8 = jnp.ones((8, _D), jnp.float32)
    sq = lax.dot_general(ones8, xc * xc, (((1,), (1,)), ((), ())),
                         preferred_element_type=jnp.float32,
                         precision=lax.Precision.HIGHEST)[0:1, :]
    colrow = (lax.broadcasted_iota(jnp.int32, (1, _WT), 1) + t * _WT)
    sq = jnp.where(colrow >= _N, _INF, sq)

    dots = lax.dot_general(xb, xc, (((1,), (1,)), ((), ())),
                           preferred_element_type=jnp.float32)
    s = (sq - 2.0 * dots).reshape(_RB, _WT // 128, 128)

    for g in range(_WT // 128):
      v = s[:, g, :]
      iv = lane_col + (t * _WT + g * 128)
      v = jnp.where(iv == rows_col, _INF, v)
      for l in range(_NLVL):
        cond = v < M[l]
        if l < _NLVL - 1:
          M[l], v = jnp.where(cond, v, M[l]), jnp.where(cond, M[l], v)
          I[l], iv = jnp.where(cond, iv, I[l]), jnp.where(cond, I[l], iv)
        else:
          M[l] = jnp.where(cond, v, M[l])
          I[l] = jnp.where(cond, iv, I[l])

  cand = jnp.concatenate(M, axis=1)
  icand = jnp.concatenate(I, axis=1)
  for k in range(_K):
    m = jnp.min(cand, axis=1, keepdims=True)
    hit = cand <= m
    amin = jnp.min(jnp.where(hit, icand, _BIGI), axis=1, keepdims=True)
    nbr_ref[:, k:k + 1] = amin
    cand = jnp.where(hit, _INF, cand)
 return _knn_body


def _knn_topk(x, xcols, off_blocks, nblk):
  return pl.pallas_call(
      _make_knn_body(off_blocks),
      grid=(nblk,),
      in_specs=[
          pl.BlockSpec((_RB, _D), lambda i, o=off_blocks: (i + o, 0)),
          pl.BlockSpec((_NTILES, _WT, _D), lambda i: (0, 0, 0)),
      ],
      out_specs=pl.BlockSpec((_RB, _K), lambda i: (i, 0)),
      out_shape=jax.ShapeDtypeStruct((nblk * _RB, _K), jnp.int32),
  )(x, xcols)


_NW = 32
_CG = 40


def _gather_src(nbr_flat, xl):
  _E = nbr_flat.shape[0]
  _EPW = _E // _NW
  _NCHUNK = _EPW // _CG
  mesh = plsc.VectorSubcoreMesh(core_axis_name="c", subcore_axis_name="s")

  @functools.partial(
      pl.kernel,
      out_type=jax.ShapeDtypeStruct((_E, _H * _C), jnp.float32),
      mesh=mesh,
      scratch_types=[
          pltpu.VMEM((2, _CG), jnp.int32),
          pltpu.VMEM((2, _CG, _H * _C), jnp.float32),
          pltpu.SemaphoreType.DMA,
          pltpu.SemaphoreType.DMA,
      ],
  )
  def gather_kernel(idx_hbm, xl_hbm, out_hbm, idx_v, rows_v, sem0, sem1):
    wid = lax.axis_index("s") * 2 + lax.axis_index("c")
    base = wid * _EPW
    sems = [sem0, sem1]

    pltpu.sync_copy(idx_hbm.at[pl.ds(base, _CG)], idx_v.at[0])
    pltpu.async_copy(xl_hbm.at[idx_v.at[0]], rows_v.at[0], sems[0])

    def body(j, _):
      for b in range(2):
        i = 2 * j + b

        @pl.when(i + 1 < _NCHUNK)
        def _start_next():
          pltpu.sync_copy(idx_hbm.at[pl.ds(base + (i + 1) * _CG, _CG)],
                          idx_v.at[1 - b])
          pltpu.async_copy(xl_hbm.at[idx_v.at[1 - b]], rows_v.at[1 - b],
                           sems[1 - b])

        @pl.when(i < _NCHUNK)
        def _drain_cur():
          pltpu.make_async_copy(xl_hbm.at[idx_v.at[b]], rows_v.at[b],
                                sems[b]).wait()
          pltpu.sync_copy(rows_v.at[b],
                          out_hbm.at[pl.ds(base + i * _CG, _CG)])

      return 0

    lax.fori_loop(0, (_NCHUNK + 1) // 2, body, 0)

  return gather_kernel(nbr_flat, xl)


_RA = 400


def _attn_body(src_ref, xr_ref, attw_ref, bias_ref, out_ref):
  src = src_ref[...]
  xr = xr_ref[...]
  xr_rep = jnp.broadcast_to(
      xr[:, None, :], (_RA, _K, _H * _C)).reshape(_RA * _K, _H * _C)
  e = src + xr_rep
  e = jnp.maximum(e, 0.2 * e)
  lg = jnp.dot(e, attw_ref[...],
               preferred_element_type=jnp.float32)
  lg3 = lg.reshape(_RA, _K, _C)
  m = jnp.max(lg3, axis=1, keepdims=True)
  p = jnp.exp(lg3 - m)
  al = p / jnp.sum(p, axis=1, keepdims=True)
  src3 = src.reshape(_RA, _K, _H * _C)
  acc = jnp.zeros((_RA, _C), jnp.float32)
  for h in range(_H):
    alh = al[:, :, h:h + 1]
    acc = acc + jnp.sum(alh * src3[:, :, h * _C:(h + 1) * _C], axis=1)
  out_ref[...] = acc * (1.0 / _H) + bias_ref[...]


def _attention(src, xr, attw, bias):
  nrows = xr.shape[0]
  return pl.pallas_call(
      _attn_body,
      grid=(nrows // _RA,),
      in_specs=[
          pl.BlockSpec((_RA * _K, _H * _C), lambda i: (i, 0)),
          pl.BlockSpec((_RA, _H * _C), lambda i: (i, 0)),
          pl.BlockSpec((_H * _C, _C), lambda i: (0, 0)),
          pl.BlockSpec((1, _C), lambda i: (0, 0)),
      ],
      out_specs=pl.BlockSpec((_RA, _C), lambda i: (i, 0)),
      out_shape=jax.ShapeDtypeStruct((nrows, _C), jnp.float32),
  )(src, xr, attw, bias.reshape(1, -1))


_PARTS = (13, 12)


def kernel(x, Wl, bl, Wr, br, att, bias):
  xl, xr = _transforms(x, Wl, bl, Wr, br)
  xcols = jnp.pad(x, ((0, _NPAD - _N), (0, 0))).reshape(_NTILES, _WT, _D)
  attw = jnp.zeros((_H * _C, _C), jnp.float32).at[
      jnp.arange(_H * _C), jnp.arange(_H * _C) // _C].set(att.reshape(-1))

  outs = []
  off = 0
  srcs = []
  for nblk in _PARTS:
    nbr = _knn_topk(x, xcols, off, nblk)
    srcs.append(_gather_src(nbr.reshape(-1), xl))
    off += nblk
  off = 0
  for nblk, src in zip(_PARTS, srcs):
    outs.append(_attention(
        src, xr[off * _RB:(off + nblk) * _RB], attw, bias))
    off += nblk
  return jnp.concatenate(outs, axis=0)
